# baseline (speedup 1.0000x reference)
import functools

import jax
import jax.numpy as jnp
from jax import lax
from jax.experimental import pallas as pl
from jax.experimental.pallas import tpu as pltpu
from jax.experimental.pallas import tpu_sc as plsc

B, C, N, RES = 8, 3, 16384, 64
REG_COEF = 0.1
TOTAL_IDX = 3 * C * B * N


def _prep_body(y_ref, pts_ref, refl_ref, idx_ref):
    y = y_ref[...]
    nx, ny, nz, dd = y[:, :, 0], y[:, :, 1], y[:, :, 2], y[:, :, 3]
    s1 = jnp.sqrt(nx * nx + ny * ny + nz * nz)
    nx, ny, nz = nx / s1, ny / s1, nz / s1
    s2 = jnp.sqrt(nx * nx + ny * ny + nz * nz)
    nx, ny, nz = nx / s2, ny / s2, nz / s2

    px = pts_ref[0]
    py = pts_ref[1]
    pz = pts_ref[2]
    bio = lax.broadcasted_iota(jnp.int32, (B, N), 0)

    for c in range(C):
        cx = nx[:, c : c + 1]
        cy = ny[:, c : c + 1]
        cz = nz[:, c : c + 1]
        cd = dd[:, c : c + 1]
        dist = px * cx + py * cy + pz * cz + cd
        rx = px - 2.0 * dist * cx
        ry = py - 2.0 * dist * cy
        rz = pz - 2.0 * dist * cz
        vx = jnp.clip(jnp.floor(rx * float(RES)), 0.0, RES - 1).astype(jnp.int32)
        vy = jnp.clip(jnp.floor(ry * float(RES)), 0.0, RES - 1).astype(jnp.int32)
        vz = jnp.clip(jnp.floor(rz * float(RES)), 0.0, RES - 1).astype(jnp.int32)
        w0 = (bio * (RES * RES * RES) + (vx * RES + vy) * RES + vz) * 3
        refl_ref[0, c] = rx
        refl_ref[1, c] = ry
        refl_ref[2, c] = rz
        idx_ref[0, c] = w0
        idx_ref[1, c] = w0 + 1
        idx_ref[2, c] = w0 + 2


def _prep(y_pred, pts):
    return pl.pallas_call(
        _prep_body,
        out_shape=(
            jax.ShapeDtypeStruct((3, C, B, N), jnp.float32),
            jax.ShapeDtypeStruct((3, C, B, N), jnp.int32),
        ),
    )(y_pred, pts)


_NC, _NS = 2, 16
_NW = _NC * _NS
_PER_W = TOTAL_IDX // _NW
_CHUNK = 4096


@functools.cache
def _make_sc_gather():
    mesh = plsc.VectorSubcoreMesh(core_axis_name="c", subcore_axis_name="s")

    @functools.partial(
        pl.kernel,
        mesh=mesh,
        out_type=jax.ShapeDtypeStruct((TOTAL_IDX,), jnp.float32),
        scratch_types=[
            pltpu.VMEM((_CHUNK,), jnp.int32),
            pltpu.VMEM((_CHUNK,), jnp.float32),
            pltpu.SemaphoreType.DMA,
        ],
    )
    def sc_gather(table_hbm, idx_hbm, out_hbm, idx_v, val_v, sem):
        wid = lax.axis_index("s") * _NC + lax.axis_index("c")
        base = wid * _PER_W

        def body(i, carry):
            off = base + i * _CHUNK
            pltpu.sync_copy(idx_hbm.at[pl.ds(off, _CHUNK)], idx_v)
            pltpu.async_copy(table_hbm.at[idx_v], val_v, sem).wait()
            pltpu.sync_copy(val_v, out_hbm.at[pl.ds(off, _CHUNK)])
            return carry

        lax.fori_loop(0, _PER_W // _CHUNK, body, 0)

    return sc_gather


def _finish_body(y_ref, refl_ref, cp_ref, out_ref):
    acc = jnp.zeros((B, N), jnp.float32)
    for c in range(C):
        dx = refl_ref[0, c] - cp_ref[0, c]
        dy = refl_ref[1, c] - cp_ref[1, c]
        dz = refl_ref[2, c] - cp_ref[2, c]
        acc = acc + jnp.sqrt(dx * dx + dy * dy + dz * dz)
    refl_total = jnp.sum(acc) * (1.0 / N)

    y = y_ref[...]
    nx, ny, nz = y[:, :, 0], y[:, :, 1], y[:, :, 2]
    s = jnp.sqrt(nx * nx + ny * ny + nz * nz)
    nx, ny, nz = nx / s, ny / s, nz / s
    acc_reg = jnp.zeros((B,), jnp.float32)
    for i in range(C):
        for k in range(C):
            dot = nx[:, i] * nx[:, k] + ny[:, i] * ny[:, k] + nz[:, i] * nz[:, k]
            if i == k:
                dot = dot - 1.0
            acc_reg = acc_reg + dot * dot
    reg = jnp.sqrt(acc_reg)
    out_ref[...] = jnp.reshape(refl_total + REG_COEF * jnp.mean(reg), (1, 1))


def _finish(y_pred, refl, cp):
    return pl.pallas_call(
        _finish_body,
        out_shape=jax.ShapeDtypeStruct((1, 1), jnp.float32),
    )(y_pred, refl, cp)


def kernel(y_pred, points, voxel_grid, voxel_grid_cp):
    pts = jnp.transpose(points, (2, 0, 1))
    refl, widx = _prep(y_pred, pts)
    table = voxel_grid_cp.reshape(-1)
    gathered = _make_sc_gather()(table, widx.reshape(-1))
    cp = gathered.reshape(3, C, B, N)
    total = _finish(y_pred, refl, cp)
    return total.reshape(1)


# speedup vs baseline: 11.8379x; 11.8379x over previous
import functools

import jax
import jax.numpy as jnp
from jax import lax
from jax.experimental import pallas as pl
from jax.experimental.pallas import tpu as pltpu
from jax.experimental.pallas import tpu_sc as plsc

B, C, N, RES = 8, 3, 16384, 64
REG_COEF = 0.1
NPTS = B * C * N
BATCH_WORDS = RES * C * RES * RES

_NC, _NS = 2, 16
_NW = _NC * _NS
PER_TILE = NPTS // _NW // 4 * 4
B_PER_SC = B // _NC
PTS_PER_BATCH = C * N
TILE_PTS = PTS_PER_BATCH // _NS
CHUNK = 1024
N_CHUNK = TILE_PTS // CHUNK


def _prep_body(y_ref, pts_ref, rx_ref, ry_ref, rz_ref, i0_ref, i1_ref, i2_ref):
    y = y_ref[...]
    nx, ny, nz, dd = y[:, :, 0], y[:, :, 1], y[:, :, 2], y[:, :, 3]
    s1 = jnp.sqrt(nx * nx + ny * ny + nz * nz)
    nx, ny, nz = nx / s1, ny / s1, nz / s1
    s2 = jnp.sqrt(nx * nx + ny * ny + nz * nz)
    nx, ny, nz = nx / s2, ny / s2, nz / s2

    px = pts_ref[0]
    py = pts_ref[1]
    pz = pts_ref[2]

    for c in range(C):
        cx = nx[:, c : c + 1]
        cy = ny[:, c : c + 1]
        cz = nz[:, c : c + 1]
        cd = dd[:, c : c + 1]
        dist = px * cx + py * cy + pz * cz + cd
        rx = px - 2.0 * dist * cx
        ry = py - 2.0 * dist * cy
        rz = pz - 2.0 * dist * cz
        vx = jnp.clip(jnp.floor(rx * float(RES)), 0.0, RES - 1).astype(jnp.int32)
        vy = jnp.clip(jnp.floor(ry * float(RES)), 0.0, RES - 1).astype(jnp.int32)
        vz = jnp.clip(jnp.floor(rz * float(RES)), 0.0, RES - 1).astype(jnp.int32)
        w0 = vx * (C * RES * RES) + vy * RES + vz
        for b in range(B):
            r = b * C + c
            rx_ref[r, :] = rx[b]
            ry_ref[r, :] = ry[b]
            rz_ref[r, :] = rz[b]
            i0_ref[r, :] = w0[b]
            i1_ref[r, :] = w0[b] + RES * RES
            i2_ref[r, :] = w0[b] + 2 * RES * RES


def _prep(y_pred, pts):
    outs = [jax.ShapeDtypeStruct((B * C, N), jnp.float32)] * 3 + [
        jax.ShapeDtypeStruct((B * C, N), jnp.int32)
    ] * 3
    return pl.pallas_call(_prep_body, out_shape=tuple(outs))(y_pred, pts)


def _sqrt16(s):
    i = lax.bitcast_convert_type(s, jnp.int32)
    i = jnp.int32(0x5F3759DF) - lax.shift_right_arithmetic(i, 1)
    y = lax.bitcast_convert_type(i, jnp.float32)
    for _ in range(3):
        y = y * (1.5 - 0.5 * s * y * y)
    return s * y


@functools.cache
def _make_sc_main():
    mesh = plsc.VectorSubcoreMesh(core_axis_name="c", subcore_axis_name="s")

    @functools.partial(
        pl.kernel,
        mesh=mesh,
        out_type=jax.ShapeDtypeStruct((_NW * 16,), jnp.float32),
        scratch_types=[
            pltpu.VMEM_SHARED((BATCH_WORDS,), jnp.float32),
            pltpu.VMEM((CHUNK,), jnp.int32),
            pltpu.VMEM((CHUNK,), jnp.int32),
            pltpu.VMEM((CHUNK,), jnp.int32),
            pltpu.VMEM((CHUNK,), jnp.float32),
            pltpu.VMEM((CHUNK,), jnp.float32),
            pltpu.VMEM((CHUNK,), jnp.float32),
            pltpu.VMEM((CHUNK,), jnp.float32),
            pltpu.VMEM((CHUNK,), jnp.float32),
            pltpu.VMEM((CHUNK,), jnp.float32),
            pltpu.VMEM((16,), jnp.float32),
            pltpu.SemaphoreType.DMA,
        ],
    )
    def sc_main(
        table_hbm, i0_hbm, i1_hbm, i2_hbm, rx_hbm, ry_hbm, rz_hbm, out_hbm,
        spmem, i0v, i1v, i2v, c0v, c1v, c2v, rxv, ryv, rzv, accv, sem,
    ):
        core = lax.axis_index("c")
        sid = lax.axis_index("s")
        stage_sz = BATCH_WORDS // _NS

        acc = jnp.zeros((16,), jnp.float32)
        for bi in range(B_PER_SC):
            b = core * B_PER_SC + bi
            pltpu.sync_copy(
                table_hbm.at[pl.ds(b * BATCH_WORDS + sid * stage_sz, stage_sz)],
                spmem.at[pl.ds(sid * stage_sz, stage_sz)],
            )
            plsc.subcore_barrier()
            for ck in range(N_CHUNK):
                off = b * PTS_PER_BATCH + sid * TILE_PTS + ck * CHUNK
                sl = pl.ds(off, CHUNK)
                pltpu.sync_copy(i0_hbm.at[sl], i0v)
                pltpu.sync_copy(i1_hbm.at[sl], i1v)
                pltpu.sync_copy(i2_hbm.at[sl], i2v)
                pltpu.sync_copy(rx_hbm.at[sl], rxv)
                pltpu.sync_copy(ry_hbm.at[sl], ryv)
                pltpu.sync_copy(rz_hbm.at[sl], rzv)
                pltpu.async_copy(spmem.at[i0v], c0v, sem).wait()
                pltpu.async_copy(spmem.at[i1v], c1v, sem).wait()
                pltpu.async_copy(spmem.at[i2v], c2v, sem).wait()

                def body(j, a):
                    vs = pl.ds(j * 16, 16)
                    dx = rxv[vs] - c0v[vs]
                    dy = ryv[vs] - c1v[vs]
                    dz = rzv[vs] - c2v[vs]
                    return a + _sqrt16(dx * dx + dy * dy + dz * dz)

                acc = lax.fori_loop(0, CHUNK // 16, body, acc)
            plsc.subcore_barrier()

        accv[...] = acc
        wid = core * _NS + sid
        pltpu.sync_copy(accv, out_hbm.at[pl.ds(wid * 16, 16)])

    return sc_main


def _finish_body(y_ref, part_ref, out_ref):
    refl_total = jnp.sum(part_ref[...]) * (1.0 / N)

    y = y_ref[...]
    nx, ny, nz = y[:, :, 0], y[:, :, 1], y[:, :, 2]
    s = jnp.sqrt(nx * nx + ny * ny + nz * nz)
    nx, ny, nz = nx / s, ny / s, nz / s
    acc_reg = jnp.zeros((B,), jnp.float32)
    for i in range(C):
        for k in range(C):
            dot = nx[:, i] * nx[:, k] + ny[:, i] * ny[:, k] + nz[:, i] * nz[:, k]
            if i == k:
                dot = dot - 1.0
            acc_reg = acc_reg + dot * dot
    reg = jnp.sqrt(acc_reg)
    out_ref[...] = jnp.reshape(refl_total + REG_COEF * jnp.mean(reg), (1, 1))


def _finish(y_pred, partials):
    return pl.pallas_call(
        _finish_body,
        out_shape=jax.ShapeDtypeStruct((1, 1), jnp.float32),
    )(y_pred, partials)


def kernel(y_pred, points, voxel_grid, voxel_grid_cp):
    pts = jnp.transpose(points, (2, 0, 1))
    rx, ry, rz, i0, i1, i2 = _prep(y_pred, pts)
    table = jnp.transpose(voxel_grid_cp, (0, 1, 4, 2, 3)).reshape(-1)
    partials = _make_sc_main()(
        table,
        i0.reshape(-1), i1.reshape(-1), i2.reshape(-1),
        rx.reshape(-1), ry.reshape(-1), rz.reshape(-1),
    )
    total = _finish(y_pred, partials)
    return total.reshape(1)


# speedup vs baseline: 13.7599x; 1.1624x over previous
import functools

import jax
import jax.numpy as jnp
from jax import lax
from jax.experimental import pallas as pl
from jax.experimental.pallas import tpu as pltpu
from jax.experimental.pallas import tpu_sc as plsc

B, C, N, RES = 8, 3, 16384, 64
REG_COEF = 0.1
NPTS = B * C * N
BATCH_WORDS = RES * C * RES * RES

_NC, _NS = 2, 16
_NW = _NC * _NS
PER_TILE = NPTS // _NW // 4 * 4
B_PER_SC = B // _NC
PTS_PER_BATCH = C * N
TILE_PTS = PTS_PER_BATCH // _NS
CHUNK = 1536
N_CHUNK = TILE_PTS // CHUNK


def _prep_body(y_ref, pts_ref, rx_ref, ry_ref, rz_ref, i0_ref, i1_ref, i2_ref):
    y = y_ref[...]
    nx, ny, nz, dd = y[:, :, 0], y[:, :, 1], y[:, :, 2], y[:, :, 3]
    s1 = jnp.sqrt(nx * nx + ny * ny + nz * nz)
    nx, ny, nz = nx / s1, ny / s1, nz / s1
    s2 = jnp.sqrt(nx * nx + ny * ny + nz * nz)
    nx, ny, nz = nx / s2, ny / s2, nz / s2

    px = pts_ref[0]
    py = pts_ref[1]
    pz = pts_ref[2]

    for c in range(C):
        cx = nx[:, c : c + 1]
        cy = ny[:, c : c + 1]
        cz = nz[:, c : c + 1]
        cd = dd[:, c : c + 1]
        dist = px * cx + py * cy + pz * cz + cd
        rx = px - 2.0 * dist * cx
        ry = py - 2.0 * dist * cy
        rz = pz - 2.0 * dist * cz
        vx = jnp.clip(jnp.floor(rx * float(RES)), 0.0, RES - 1).astype(jnp.int32)
        vy = jnp.clip(jnp.floor(ry * float(RES)), 0.0, RES - 1).astype(jnp.int32)
        vz = jnp.clip(jnp.floor(rz * float(RES)), 0.0, RES - 1).astype(jnp.int32)
        w0 = vx * (C * RES * RES) + vy * RES + vz
        for b in range(B):
            r = b * C + c
            rx_ref[r, :] = rx[b]
            ry_ref[r, :] = ry[b]
            rz_ref[r, :] = rz[b]
            i0_ref[r, :] = w0[b]
            i1_ref[r, :] = w0[b] + RES * RES
            i2_ref[r, :] = w0[b] + 2 * RES * RES


def _prep(y_pred, pts):
    outs = [jax.ShapeDtypeStruct((B * C, N), jnp.float32)] * 3 + [
        jax.ShapeDtypeStruct((B * C, N), jnp.int32)
    ] * 3
    return pl.pallas_call(_prep_body, out_shape=tuple(outs))(y_pred, pts)


def _sqrt16(s):
    i = lax.bitcast_convert_type(s, jnp.int32)
    i = jnp.int32(0x5F3759DF) - lax.shift_right_arithmetic(i, 1)
    y = lax.bitcast_convert_type(i, jnp.float32)
    for _ in range(3):
        y = y * (1.5 - 0.5 * s * y * y)
    return s * y


@functools.cache
def _make_sc_main():
    mesh = plsc.VectorSubcoreMesh(core_axis_name="c", subcore_axis_name="s")

    in_bufs = [
        pltpu.VMEM((TILE_PTS,), jnp.int32) if k < 3 else pltpu.VMEM((TILE_PTS,), jnp.float32)
        for k in range(6)
    ]
    g_bufs = [pltpu.VMEM((CHUNK,), jnp.float32) for _ in range(2) for _ in range(3)]

    @functools.partial(
        pl.kernel,
        mesh=mesh,
        out_type=jax.ShapeDtypeStruct((_NW * 16,), jnp.float32),
        scratch_types=[pltpu.VMEM_SHARED((2 * BATCH_WORDS,), jnp.float32)]
        + in_bufs
        + g_bufs
        + [
            pltpu.VMEM((16,), jnp.float32),
            pltpu.SemaphoreType.DMA,
            pltpu.SemaphoreType.DMA,
            pltpu.SemaphoreType.DMA,
        ],
    )
    def sc_main(
        table_hbm, i0_hbm, i1_hbm, i2_hbm, rx_hbm, ry_hbm, rz_hbm, out_hbm,
        spmem,
        a_i0, a_i1, a_i2, a_rx, a_ry, a_rz,
        g0c0, g0c1, g0c2, g1c0, g1c1, g1c2,
        accv, sem_stage, sem_in, sem_g,
    ):
        core = lax.axis_index("c")
        sid = lax.axis_index("s")
        stage_sz = BATCH_WORDS // _NS
        inset = (a_i0, a_i1, a_i2, a_rx, a_ry, a_rz)
        gsets = [(g0c0, g0c1, g0c2), (g1c0, g1c1, g1c2)]

        def sync_stage(bi):
            b = core * B_PER_SC + bi
            half = stage_sz // 2
            for h in range(2):
                pltpu.sync_copy(
                    table_hbm.at[
                        pl.ds(b * BATCH_WORDS + sid * stage_sz + h * half, half)
                    ],
                    spmem.at[
                        pl.ds(
                            (bi % 2) * BATCH_WORDS + sid * stage_sz + h * half, half
                        )
                    ],
                )

        def fire_in(bi):
            b = core * B_PER_SC + bi
            off = b * PTS_PER_BATCH + sid * TILE_PTS
            sl = pl.ds(off, TILE_PTS)
            hb = (i0_hbm, i1_hbm, i2_hbm, rx_hbm, ry_hbm, rz_hbm)
            return [pltpu.async_copy(h.at[sl], v, sem_in) for h, v in zip(hb, inset)]

        acc = jnp.zeros((16,), jnp.float32)
        sync_stage(0)
        in_d = fire_in(0)
        plsc.subcore_barrier()
        for bi in range(B_PER_SC):
            for d in in_d:
                d.wait()
            i0v, i1v, i2v, rxv, ryv, rzv = inset
            sp = spmem.at[pl.ds((bi % 2) * BATCH_WORDS, BATCH_WORDS)]
            g_d = []
            for ck in range(N_CHUNK):
                cs = pl.ds(ck * CHUNK, CHUNK)
                c0v, c1v, c2v = gsets[ck % 2]
                g_d.append([
                    pltpu.async_copy(sp.at[i0v.at[cs]], c0v, sem_g),
                    pltpu.async_copy(sp.at[i1v.at[cs]], c1v, sem_g),
                    pltpu.async_copy(sp.at[i2v.at[cs]], c2v, sem_g),
                ])
            if bi + 1 < B_PER_SC:
                sync_stage(bi + 1)
            for ck in range(N_CHUNK):
                for d in g_d[ck]:
                    d.wait()
                c0v, c1v, c2v = gsets[ck % 2]

                def body(j, a, _ck=ck, _c0=c0v, _c1=c1v, _c2=c2v,
                         _rx=rxv, _ry=ryv, _rz=rzv):
                    vs = pl.ds(j * 16, 16)
                    ps = pl.ds(_ck * CHUNK + j * 16, 16)
                    dx = _rx[ps] - _c0[vs]
                    dy = _ry[ps] - _c1[vs]
                    dz = _rz[ps] - _c2[vs]
                    return a + _sqrt16(dx * dx + dy * dy + dz * dz)

                acc = lax.fori_loop(0, CHUNK // 16, body, acc)
            if bi + 1 < B_PER_SC:
                in_d = fire_in(bi + 1)
            plsc.subcore_barrier()

        accv[...] = acc
        wid = core * _NS + sid
        pltpu.sync_copy(accv, out_hbm.at[pl.ds(wid * 16, 16)])

    return sc_main


def _finish_body(y_ref, part_ref, out_ref):
    refl_total = jnp.sum(part_ref[...]) * (1.0 / N)

    y = y_ref[...]
    nx, ny, nz = y[:, :, 0], y[:, :, 1], y[:, :, 2]
    s = jnp.sqrt(nx * nx + ny * ny + nz * nz)
    nx, ny, nz = nx / s, ny / s, nz / s
    acc_reg = jnp.zeros((B,), jnp.float32)
    for i in range(C):
        for k in range(C):
            dot = nx[:, i] * nx[:, k] + ny[:, i] * ny[:, k] + nz[:, i] * nz[:, k]
            if i == k:
                dot = dot - 1.0
            acc_reg = acc_reg + dot * dot
    reg = jnp.sqrt(acc_reg)
    out_ref[...] = jnp.reshape(refl_total + REG_COEF * jnp.mean(reg), (1, 1))


def _finish(y_pred, partials):
    return pl.pallas_call(
        _finish_body,
        out_shape=jax.ShapeDtypeStruct((1, 1), jnp.float32),
    )(y_pred, partials)


def kernel(y_pred, points, voxel_grid, voxel_grid_cp):
    pts = jnp.transpose(points, (2, 0, 1))
    rx, ry, rz, i0, i1, i2 = _prep(y_pred, pts)
    table = jnp.transpose(voxel_grid_cp, (0, 1, 4, 2, 3)).reshape(-1)
    partials = _make_sc_main()(
        table,
        i0.reshape(-1), i1.reshape(-1), i2.reshape(-1),
        rx.reshape(-1), ry.reshape(-1), rz.reshape(-1),
    )
    total = _finish(y_pred, partials)
    return total.reshape(1)


# speedup vs baseline: 14.6221x; 1.0627x over previous
import functools

import jax
import jax.numpy as jnp
from jax import lax
from jax.experimental import pallas as pl
from jax.experimental.pallas import tpu as pltpu
from jax.experimental.pallas import tpu_sc as plsc

B, C, N, RES = 8, 3, 16384, 64
REG_COEF = 0.1
NPTS = B * C * N
BATCH_WORDS = RES * C * RES * RES

_NC, _NS = 2, 16
_NW = _NC * _NS
PER_TILE = NPTS // _NW // 4 * 4
B_PER_SC = B // _NC
PTS_PER_BATCH = C * N
TILE_PTS = PTS_PER_BATCH // _NS
CHUNK = 1536
N_CHUNK = TILE_PTS // CHUNK


def _prep_body(y_ref, pts_ref, rx_ref, ry_ref, rz_ref, i0_ref, i1_ref, i2_ref):
    y = y_ref[...]
    nx, ny, nz, dd = y[:, :, 0], y[:, :, 1], y[:, :, 2], y[:, :, 3]
    s1 = jnp.sqrt(nx * nx + ny * ny + nz * nz)
    nx, ny, nz = nx / s1, ny / s1, nz / s1
    s2 = jnp.sqrt(nx * nx + ny * ny + nz * nz)
    nx, ny, nz = nx / s2, ny / s2, nz / s2

    px = pts_ref[0]
    py = pts_ref[1]
    pz = pts_ref[2]

    for c in range(C):
        cx = nx[:, c : c + 1]
        cy = ny[:, c : c + 1]
        cz = nz[:, c : c + 1]
        cd = dd[:, c : c + 1]
        dist = px * cx + py * cy + pz * cz + cd
        rx = px - 2.0 * dist * cx
        ry = py - 2.0 * dist * cy
        rz = pz - 2.0 * dist * cz
        vx = jnp.clip(jnp.floor(rx * float(RES)), 0.0, RES - 1).astype(jnp.int32)
        vy = jnp.clip(jnp.floor(ry * float(RES)), 0.0, RES - 1).astype(jnp.int32)
        vz = jnp.clip(jnp.floor(rz * float(RES)), 0.0, RES - 1).astype(jnp.int32)
        w0 = vx * (C * RES * RES) + vy * RES + vz
        for b in range(B):
            sl = pl.ds((b * C + c) * N, N)
            rx_ref[sl] = rx[b]
            ry_ref[sl] = ry[b]
            rz_ref[sl] = rz[b]
            i0_ref[sl] = w0[b]
            i1_ref[sl] = w0[b] + RES * RES
            i2_ref[sl] = w0[b] + 2 * RES * RES


def _prep(y_pred, pts):
    outs = [jax.ShapeDtypeStruct((NPTS,), jnp.float32)] * 3 + [
        jax.ShapeDtypeStruct((NPTS,), jnp.int32)
    ] * 3
    return pl.pallas_call(_prep_body, out_shape=tuple(outs))(y_pred, pts)


def _sqrt16(s):
    i = lax.bitcast_convert_type(s, jnp.int32)
    i = jnp.int32(0x5F3759DF) - lax.shift_right_arithmetic(i, 1)
    y = lax.bitcast_convert_type(i, jnp.float32)
    for _ in range(3):
        y = y * (1.5 - 0.5 * s * y * y)
    return s * y


@functools.cache
def _make_sc_main():
    mesh = plsc.VectorSubcoreMesh(core_axis_name="c", subcore_axis_name="s")

    in_bufs = [
        pltpu.VMEM((TILE_PTS,), jnp.int32) if k < 3 else pltpu.VMEM((TILE_PTS,), jnp.float32)
        for k in range(6)
    ]
    g_bufs = [pltpu.VMEM((CHUNK,), jnp.float32) for _ in range(2) for _ in range(3)]

    @functools.partial(
        pl.kernel,
        mesh=mesh,
        out_type=jax.ShapeDtypeStruct((_NW * 16,), jnp.float32),
        scratch_types=[pltpu.VMEM_SHARED((2 * BATCH_WORDS,), jnp.float32)]
        + in_bufs
        + g_bufs
        + [
            pltpu.VMEM((16,), jnp.float32),
            pltpu.SemaphoreType.DMA,
            pltpu.SemaphoreType.DMA,
            pltpu.SemaphoreType.DMA,
        ],
    )
    def sc_main(
        table_hbm, i0_hbm, i1_hbm, i2_hbm, rx_hbm, ry_hbm, rz_hbm, out_hbm,
        spmem,
        a_i0, a_i1, a_i2, a_rx, a_ry, a_rz,
        g0c0, g0c1, g0c2, g1c0, g1c1, g1c2,
        accv, sem_stage, sem_in, sem_g,
    ):
        core = lax.axis_index("c")
        sid = lax.axis_index("s")
        stage_sz = BATCH_WORDS // _NS
        inset = (a_i0, a_i1, a_i2, a_rx, a_ry, a_rz)
        gsets = [(g0c0, g0c1, g0c2), (g1c0, g1c1, g1c2)]

        def sync_stage(bi):
            b = core * B_PER_SC + bi
            half = stage_sz // 2
            for h in range(2):
                pltpu.sync_copy(
                    table_hbm.at[
                        pl.ds(b * BATCH_WORDS + sid * stage_sz + h * half, half)
                    ],
                    spmem.at[
                        pl.ds(
                            (bi % 2) * BATCH_WORDS + sid * stage_sz + h * half, half
                        )
                    ],
                )

        def fire_in(bi):
            b = core * B_PER_SC + bi
            off = b * PTS_PER_BATCH + sid * TILE_PTS
            sl = pl.ds(off, TILE_PTS)
            hb = (i0_hbm, i1_hbm, i2_hbm, rx_hbm, ry_hbm, rz_hbm)
            return [pltpu.async_copy(h.at[sl], v, sem_in) for h, v in zip(hb, inset)]

        acc = jnp.zeros((16,), jnp.float32)
        sync_stage(0)
        in_d = fire_in(0)
        plsc.subcore_barrier()
        for bi in range(B_PER_SC):
            for d in in_d:
                d.wait()
            i0v, i1v, i2v, rxv, ryv, rzv = inset
            sp = spmem.at[pl.ds((bi % 2) * BATCH_WORDS, BATCH_WORDS)]
            g_d = []
            for ck in range(N_CHUNK):
                cs = pl.ds(ck * CHUNK, CHUNK)
                c0v, c1v, c2v = gsets[ck % 2]
                g_d.append([
                    pltpu.async_copy(sp.at[i0v.at[cs]], c0v, sem_g),
                    pltpu.async_copy(sp.at[i1v.at[cs]], c1v, sem_g),
                    pltpu.async_copy(sp.at[i2v.at[cs]], c2v, sem_g),
                ])
            if bi + 1 < B_PER_SC:
                sync_stage(bi + 1)
            for ck in range(N_CHUNK):
                for d in g_d[ck]:
                    d.wait()
                c0v, c1v, c2v = gsets[ck % 2]

                def body(j, a, _ck=ck, _c0=c0v, _c1=c1v, _c2=c2v,
                         _rx=rxv, _ry=ryv, _rz=rzv):
                    vs = pl.ds(j * 16, 16)
                    ps = pl.ds(_ck * CHUNK + j * 16, 16)
                    dx = _rx[ps] - _c0[vs]
                    dy = _ry[ps] - _c1[vs]
                    dz = _rz[ps] - _c2[vs]
                    return a + _sqrt16(dx * dx + dy * dy + dz * dz)

                acc = lax.fori_loop(0, CHUNK // 16, body, acc)
            if bi + 1 < B_PER_SC:
                in_d = fire_in(bi + 1)
            plsc.subcore_barrier()

        accv[...] = acc
        wid = core * _NS + sid
        pltpu.sync_copy(accv, out_hbm.at[pl.ds(wid * 16, 16)])

    return sc_main


def _finish_body(y_ref, part_ref, out_ref):
    refl_total = jnp.sum(part_ref[...]) * (1.0 / N)

    y = y_ref[...]
    nx, ny, nz = y[:, :, 0], y[:, :, 1], y[:, :, 2]
    s = jnp.sqrt(nx * nx + ny * ny + nz * nz)
    nx, ny, nz = nx / s, ny / s, nz / s
    acc_reg = jnp.zeros((B,), jnp.float32)
    for i in range(C):
        for k in range(C):
            dot = nx[:, i] * nx[:, k] + ny[:, i] * ny[:, k] + nz[:, i] * nz[:, k]
            if i == k:
                dot = dot - 1.0
            acc_reg = acc_reg + dot * dot
    reg = jnp.sqrt(acc_reg)
    out_ref[...] = jnp.reshape(refl_total + REG_COEF * jnp.mean(reg), (1, 1))


def _finish(y_pred, partials):
    return pl.pallas_call(
        _finish_body,
        out_shape=jax.ShapeDtypeStruct((1, 1), jnp.float32),
    )(y_pred, partials)


def kernel(y_pred, points, voxel_grid, voxel_grid_cp):
    pts = jnp.transpose(points, (2, 0, 1))
    rx, ry, rz, i0, i1, i2 = _prep(y_pred, pts)
    table = jnp.transpose(voxel_grid_cp, (0, 1, 4, 2, 3)).reshape(-1)
    partials = _make_sc_main()(table, i0, i1, i2, rx, ry, rz)
    total = _finish(y_pred, partials)
    return total.reshape(1)


# speedup vs baseline: 14.6434x; 1.0015x over previous
import functools

import jax
import jax.numpy as jnp
from jax import lax
from jax.experimental import pallas as pl
from jax.experimental.pallas import tpu as pltpu
from jax.experimental.pallas import tpu_sc as plsc

B, C, N, RES = 8, 3, 16384, 64
REG_COEF = 0.1
NPTS = B * C * N
BATCH_WORDS = RES * C * RES * RES

_NC, _NS = 2, 16
_NW = _NC * _NS
PER_TILE = NPTS // _NW // 4 * 4
B_PER_SC = B // _NC
PTS_PER_BATCH = C * N
TILE_PTS = PTS_PER_BATCH // _NS
CHUNK = 768
N_CHUNK = TILE_PTS // CHUNK


def _prep_body(y_ref, pts_ref, rx_ref, ry_ref, rz_ref, i0_ref, i1_ref, i2_ref):
    y = y_ref[...]
    nx, ny, nz, dd = y[:, :, 0], y[:, :, 1], y[:, :, 2], y[:, :, 3]
    s1 = jnp.sqrt(nx * nx + ny * ny + nz * nz)
    nx, ny, nz = nx / s1, ny / s1, nz / s1
    s2 = jnp.sqrt(nx * nx + ny * ny + nz * nz)
    nx, ny, nz = nx / s2, ny / s2, nz / s2

    px = pts_ref[0]
    py = pts_ref[1]
    pz = pts_ref[2]

    for c in range(C):
        cx = nx[:, c : c + 1]
        cy = ny[:, c : c + 1]
        cz = nz[:, c : c + 1]
        cd = dd[:, c : c + 1]
        dist = px * cx + py * cy + pz * cz + cd
        rx = px - 2.0 * dist * cx
        ry = py - 2.0 * dist * cy
        rz = pz - 2.0 * dist * cz
        vx = jnp.clip(jnp.floor(rx * float(RES)), 0.0, RES - 1).astype(jnp.int32)
        vy = jnp.clip(jnp.floor(ry * float(RES)), 0.0, RES - 1).astype(jnp.int32)
        vz = jnp.clip(jnp.floor(rz * float(RES)), 0.0, RES - 1).astype(jnp.int32)
        w0 = vx * (C * RES * RES) + vy * RES + vz
        for b in range(B):
            sl = pl.ds((b * C + c) * N, N)
            rx_ref[sl] = rx[b]
            ry_ref[sl] = ry[b]
            rz_ref[sl] = rz[b]
            i0_ref[sl] = w0[b]
            i1_ref[sl] = w0[b] + RES * RES
            i2_ref[sl] = w0[b] + 2 * RES * RES


def _prep(y_pred, pts):
    outs = [jax.ShapeDtypeStruct((NPTS,), jnp.float32)] * 3 + [
        jax.ShapeDtypeStruct((NPTS,), jnp.int32)
    ] * 3
    return pl.pallas_call(_prep_body, out_shape=tuple(outs))(y_pred, pts)


def _sqrt16(s):
    i = lax.bitcast_convert_type(s, jnp.int32)
    i = jnp.int32(0x5F3759DF) - lax.shift_right_arithmetic(i, 1)
    y = lax.bitcast_convert_type(i, jnp.float32)
    for _ in range(3):
        y = y * (1.5 - 0.5 * s * y * y)
    return s * y


@functools.cache
def _make_sc_main():
    mesh = plsc.VectorSubcoreMesh(core_axis_name="c", subcore_axis_name="s")

    in_bufs = [
        pltpu.VMEM((TILE_PTS,), jnp.int32) if k < 3 else pltpu.VMEM((TILE_PTS,), jnp.float32)
        for k in range(6)
    ]
    g_bufs = [
        pltpu.VMEM((CHUNK,), jnp.float32) for _ in range(N_CHUNK) for _ in range(3)
    ]

    @functools.partial(
        pl.kernel,
        mesh=mesh,
        out_type=jax.ShapeDtypeStruct((_NW * 16,), jnp.float32),
        scratch_types=[pltpu.VMEM_SHARED((2 * BATCH_WORDS,), jnp.float32)]
        + in_bufs
        + g_bufs
        + [
            pltpu.VMEM((16,), jnp.float32),
            pltpu.SemaphoreType.DMA,
            pltpu.SemaphoreType.DMA,
            pltpu.SemaphoreType.DMA,
        ],
    )
    def sc_main(
        table_hbm, i0_hbm, i1_hbm, i2_hbm, rx_hbm, ry_hbm, rz_hbm, out_hbm,
        spmem,
        a_i0, a_i1, a_i2, a_rx, a_ry, a_rz,
        g0c0, g0c1, g0c2, g1c0, g1c1, g1c2,
        g2c0, g2c1, g2c2, g3c0, g3c1, g3c2,
        accv, sem_stage, sem_in, sem_g,
    ):
        core = lax.axis_index("c")
        sid = lax.axis_index("s")
        stage_sz = BATCH_WORDS // _NS
        inset = (a_i0, a_i1, a_i2, a_rx, a_ry, a_rz)
        gsets = [(g0c0, g0c1, g0c2), (g1c0, g1c1, g1c2),
                 (g2c0, g2c1, g2c2), (g3c0, g3c1, g3c2)]

        def sync_stage(bi):
            b = core * B_PER_SC + bi
            half = stage_sz // 2
            for h in range(2):
                pltpu.sync_copy(
                    table_hbm.at[
                        pl.ds(b * BATCH_WORDS + sid * stage_sz + h * half, half)
                    ],
                    spmem.at[
                        pl.ds(
                            (bi % 2) * BATCH_WORDS + sid * stage_sz + h * half, half
                        )
                    ],
                )

        def fire_in(bi):
            b = core * B_PER_SC + bi
            off = b * PTS_PER_BATCH + sid * TILE_PTS
            sl = pl.ds(off, TILE_PTS)
            hb = (i0_hbm, i1_hbm, i2_hbm, rx_hbm, ry_hbm, rz_hbm)
            return [pltpu.async_copy(h.at[sl], v, sem_in) for h, v in zip(hb, inset)]

        acc = jnp.zeros((16,), jnp.float32)
        sync_stage(0)
        in_d = fire_in(0)
        plsc.subcore_barrier()
        for bi in range(B_PER_SC):
            for d in in_d:
                d.wait()
            i0v, i1v, i2v, rxv, ryv, rzv = inset
            sp = spmem.at[pl.ds((bi % 2) * BATCH_WORDS, BATCH_WORDS)]
            g_d = []
            for ck in range(N_CHUNK):
                cs = pl.ds(ck * CHUNK, CHUNK)
                c0v, c1v, c2v = gsets[ck]
                g_d.append([
                    pltpu.async_copy(sp.at[i0v.at[cs]], c0v, sem_g),
                    pltpu.async_copy(sp.at[i1v.at[cs]], c1v, sem_g),
                    pltpu.async_copy(sp.at[i2v.at[cs]], c2v, sem_g),
                ])
            if bi + 1 < B_PER_SC:
                sync_stage(bi + 1)
            for ck in range(N_CHUNK):
                for d in g_d[ck]:
                    d.wait()
                c0v, c1v, c2v = gsets[ck]

                def body(j, a, _ck=ck, _c0=c0v, _c1=c1v, _c2=c2v,
                         _rx=rxv, _ry=ryv, _rz=rzv):
                    vs = pl.ds(j * 16, 16)
                    ps = pl.ds(_ck * CHUNK + j * 16, 16)
                    dx = _rx[ps] - _c0[vs]
                    dy = _ry[ps] - _c1[vs]
                    dz = _rz[ps] - _c2[vs]
                    return a + _sqrt16(dx * dx + dy * dy + dz * dz)

                acc = lax.fori_loop(0, CHUNK // 16, body, acc)
            if bi + 1 < B_PER_SC:
                in_d = fire_in(bi + 1)
            plsc.subcore_barrier()

        accv[...] = acc
        wid = core * _NS + sid
        pltpu.sync_copy(accv, out_hbm.at[pl.ds(wid * 16, 16)])

    return sc_main


def _finish_body(y_ref, part_ref, out_ref):
    refl_total = jnp.sum(part_ref[...]) * (1.0 / N)

    y = y_ref[...]
    nx, ny, nz = y[:, :, 0], y[:, :, 1], y[:, :, 2]
    s = jnp.sqrt(nx * nx + ny * ny + nz * nz)
    nx, ny, nz = nx / s, ny / s, nz / s
    acc_reg = jnp.zeros((B,), jnp.float32)
    for i in range(C):
        for k in range(C):
            dot = nx[:, i] * nx[:, k] + ny[:, i] * ny[:, k] + nz[:, i] * nz[:, k]
            if i == k:
                dot = dot - 1.0
            acc_reg = acc_reg + dot * dot
    reg = jnp.sqrt(acc_reg)
    out_ref[...] = jnp.reshape(refl_total + REG_COEF * jnp.mean(reg), (1, 1))


def _finish(y_pred, partials):
    return pl.pallas_call(
        _finish_body,
        out_shape=jax.ShapeDtypeStruct((1, 1), jnp.float32),
    )(y_pred, partials)


def kernel(y_pred, points, voxel_grid, voxel_grid_cp):
    pts = jnp.transpose(points, (2, 0, 1))
    rx, ry, rz, i0, i1, i2 = _prep(y_pred, pts)
    table = jnp.transpose(voxel_grid_cp, (0, 1, 4, 2, 3)).reshape(-1)
    partials = _make_sc_main()(table, i0, i1, i2, rx, ry, rz)
    total = _finish(y_pred, partials)
    return total.reshape(1)
